# baseline (speedup 1.0000x reference)
import functools
import math

import jax
import jax.numpy as jnp
from jax import lax
from jax.experimental import pallas as pl
from jax.experimental.pallas import tpu as pltpu
from jax.experimental.pallas import tpu_sc as plsc

K = 8
NPTS = 10000
NRAY = 8
NCAND = 16
L = 16
NWORK = 32
BIG = 3.0e38
EPS = 1e-05
CONE2_LOOSE = 0.7498
ENC_BASE = 1.0e16
ENC_STEP = 1.1e9


def _sc_search_body(pxh, pyh, pzh, oxh, oyh, ozh, rdxh, rdyh, rdzh, c0h,
                    oidx_h, ogx_h, ogy_h, ogz_h,
                    px_v, py_v, pz_v, ox_v, oy_v, oz_v,
                    rdx_v, rdy_v, rdz_v, c0_v,
                    oi_v, ogx_v, ogy_v, ogz_v):
    nc = plsc.get_sparse_core_info().num_cores
    wid = lax.axis_index("s") * nc + lax.axis_index("c")
    f0 = wid * 16
    r0 = wid * 128

    pltpu.sync_copy(pxh, px_v)
    pltpu.sync_copy(pyh, py_v)
    pltpu.sync_copy(pzh, pz_v)
    pltpu.sync_copy(oxh.at[pl.ds(f0, 16)], ox_v)
    pltpu.sync_copy(oyh.at[pl.ds(f0, 16)], oy_v)
    pltpu.sync_copy(ozh.at[pl.ds(f0, 16)], oz_v)
    pltpu.sync_copy(rdxh.at[pl.ds(r0, 128)], rdx_v)
    pltpu.sync_copy(rdyh.at[pl.ds(r0, 128)], rdy_v)
    pltpu.sync_copy(rdzh.at[pl.ds(r0, 128)], rdz_v)
    pltpu.sync_copy(c0h.at[pl.ds(r0, 128)], c0_v)

    ila = lax.iota(jnp.int32, L)

    def origin_body(fl, carry):
        ox = ox_v[fl]
        oy = oy_v[fl]
        oz = oz_v[fl]
        rdxs = [rdx_v[fl * NRAY + r] for r in range(NRAY)]
        rdys = [rdy_v[fl * NRAY + r] for r in range(NRAY)]
        rdzs = [rdz_v[fl * NRAY + r] for r in range(NRAY)]
        c0s = [c0_v[fl * NRAY + r] for r in range(NRAY)]

        t0 = tuple(jnp.full((L,), BIG, jnp.float32) for _ in range(NRAY))
        ti0 = tuple(jnp.zeros((L,), jnp.int32) for _ in range(NRAY))

        UNROLL = 5

        def chunk_body(jo, tcarry):
            ts = list(tcarry[0])
            tis = list(tcarry[1])
            for u in range(UNROLL):
                base = pl.multiple_of((jo * UNROLL + u) * L, L)
                px = px_v[pl.ds(base, L)]
                py = py_v[pl.ds(base, L)]
                pz = pz_v[pl.ds(base, L)]
                ddx = px - ox
                ddy = py - oy
                ddz = pz - oz
                d2 = ddx * ddx + ddy * ddy + ddz * ddz
                thr2 = CONE2_LOOSE * d2
                idxi = ila + base
                enc = ENC_BASE + idxi.astype(jnp.float32) * ENC_STEP
                for r in range(NRAY):
                    dot = rdxs[r] * px + rdys[r] * py + rdzs[r] * pz - c0s[r]
                    t2 = dot * dot
                    s = d2 - t2
                    keep = (dot >= 0.0) & (t2 >= thr2)
                    val = jnp.where(keep, s, enc)
                    vk, vi = plsc.sort_key_val(val, idxi, descending=True)
                    cmp = ts[r] <= vk
                    tn = jnp.where(cmp, ts[r], vk)
                    tin = jnp.where(cmp, tis[r], vi)
                    sk, si = plsc.sort_key_val(tn, tin, descending=False)
                    ts[r] = sk
                    tis[r] = si
            return (tuple(ts), tuple(tis))

        ts, tis = lax.fori_loop(0, NPTS // L // UNROLL, chunk_body, (t0, ti0))

        for r in range(NRAY):
            row = fl * NRAY + r
            oi_v[row] = tis[r]
            ogx_v[row] = plsc.load_gather(px_v, [tis[r]])
            ogy_v[row] = plsc.load_gather(py_v, [tis[r]])
            ogz_v[row] = plsc.load_gather(pz_v, [tis[r]])
        return carry

    lax.fori_loop(0, 16, origin_body, 0)

    pltpu.sync_copy(oi_v, oidx_h.at[pl.ds(r0, 128)])
    pltpu.sync_copy(ogx_v, ogx_h.at[pl.ds(r0, 128)])
    pltpu.sync_copy(ogy_v, ogy_h.at[pl.ds(r0, 128)])
    pltpu.sync_copy(ogz_v, ogz_h.at[pl.ds(r0, 128)])


def _rescore_body(ro_ref, rd_ref, ci_ref, cx_ref, cy_ref, cz_ref,
                  dist_ref, idx_ref, gx_ref, gy_ref, gz_ref):
    ox = ro_ref[:, 0:1]
    oy = ro_ref[:, 1:2]
    oz = ro_ref[:, 2:3]
    dx = cx_ref[...] - ox
    dy = cy_ref[...] - oy
    dz = cz_ref[...] - oz
    dn = jnp.sqrt(dx * dx + dy * dy + dz * dz)
    m = jnp.maximum(dn, 1e-12)
    ux = dx / m
    uy = dy / m
    uz = dz / m
    cos = rd_ref[:, 0:1] * ux + rd_ref[:, 1:2] * uy + rd_ref[:, 2:3] * uz
    om = 1.0 - cos * cos
    sin = jnp.sqrt(jnp.maximum(om, 1e-12))
    proj = sin * dn
    val = jnp.where(cos < 0.866, 1e8, proj)

    ci = ci_ref[...]
    cx = cx_ref[...]
    cy = cy_ref[...]
    cz = cz_ref[...]
    mvs, ams, gxs, gys, gzs = [], [], [], [], []
    for _ in range(K):
        mv = jnp.min(val, axis=1, keepdims=True)
        eqm = val == mv
        am = jnp.min(jnp.where(eqm, ci, NPTS), axis=1, keepdims=True)
        sel = eqm & (ci == am)
        gxs.append(jnp.min(jnp.where(sel, cx, BIG), axis=1, keepdims=True))
        gys.append(jnp.min(jnp.where(sel, cy, BIG), axis=1, keepdims=True))
        gzs.append(jnp.min(jnp.where(sel, cz, BIG), axis=1, keepdims=True))
        mvs.append(mv)
        ams.append(am)
        val = jnp.where(sel, BIG, val)

    dist_ref[...] = jnp.concatenate(mvs, axis=1)
    idx_ref[...] = jnp.concatenate(ams, axis=1)
    gx_ref[...] = jnp.concatenate(gxs, axis=1) - ox
    gy_ref[...] = jnp.concatenate(gys, axis=1) - oy
    gz_ref[...] = jnp.concatenate(gzs, axis=1) - oz


def kernel(ray_o, ray_d, pts):
    FTR = ray_o.shape[0]
    nrays = FTR * NRAY

    rdn = ray_d / jnp.maximum(jnp.linalg.norm(ray_d, axis=-1, keepdims=True), 1e-12)
    rdf = rdn.reshape(nrays, 3)
    ro_ray = jnp.repeat(ray_o, NRAY, axis=0)
    c0 = jnp.sum(rdf * ro_ray, axis=-1)

    px = pts[:, 0]
    py = pts[:, 1]
    pz = pts[:, 2]
    oxs = jnp.broadcast_to(ray_o[:, 0:1], (FTR, L))
    oys = jnp.broadcast_to(ray_o[:, 1:2], (FTR, L))
    ozs = jnp.broadcast_to(ray_o[:, 2:3], (FTR, L))
    rdxs = jnp.broadcast_to(rdf[:, 0:1], (nrays, L))
    rdys = jnp.broadcast_to(rdf[:, 1:2], (nrays, L))
    rdzs = jnp.broadcast_to(rdf[:, 2:3], (nrays, L))
    c0s = jnp.broadcast_to(c0[:, None], (nrays, L))

    mesh = plsc.VectorSubcoreMesh(core_axis_name="c", subcore_axis_name="s")
    sc_search = pl.kernel(
        _sc_search_body,
        out_type=[
            jax.ShapeDtypeStruct((nrays, NCAND), jnp.int32),
            jax.ShapeDtypeStruct((nrays, NCAND), jnp.float32),
            jax.ShapeDtypeStruct((nrays, NCAND), jnp.float32),
            jax.ShapeDtypeStruct((nrays, NCAND), jnp.float32),
        ],
        mesh=mesh,
        compiler_params=pltpu.CompilerParams(
            needs_layout_passes=False, use_tc_tiling_on_sc=False),
        scratch_types=[
            pltpu.VMEM((NPTS,), jnp.float32),
            pltpu.VMEM((NPTS,), jnp.float32),
            pltpu.VMEM((NPTS,), jnp.float32),
            pltpu.VMEM((16, L), jnp.float32),
            pltpu.VMEM((16, L), jnp.float32),
            pltpu.VMEM((16, L), jnp.float32),
            pltpu.VMEM((128, L), jnp.float32),
            pltpu.VMEM((128, L), jnp.float32),
            pltpu.VMEM((128, L), jnp.float32),
            pltpu.VMEM((128, L), jnp.float32),
            pltpu.VMEM((128, NCAND), jnp.int32),
            pltpu.VMEM((128, NCAND), jnp.float32),
            pltpu.VMEM((128, NCAND), jnp.float32),
            pltpu.VMEM((128, NCAND), jnp.float32),
        ],
    )
    cidx, cgx, cgy, cgz = sc_search(px, py, pz, oxs, oys, ozs,
                                    rdxs, rdys, rdzs, c0s)

    ro4 = jnp.pad(ro_ray, ((0, 0), (0, 1)))
    rd4 = jnp.pad(rdf, ((0, 0), (0, 1)))
    oshape = jax.ShapeDtypeStruct((nrays, K), jnp.float32)
    dist, idx, gx, gy, gz = pl.pallas_call(
        _rescore_body,
        out_shape=(oshape, jax.ShapeDtypeStruct((nrays, K), jnp.int32),
                   oshape, oshape, oshape),
    )(ro4, rd4, cidx, cgx, cgy, cgz)

    dist = dist.reshape(FTR, NRAY, K)
    idx = idx.reshape(FTR, NRAY, K)
    gx = gx.reshape(FTR, NRAY, K)
    gy = gy.reshape(FTR, NRAY, K)
    gz = gz.reshape(FTR, NRAY, K)

    npd = jnp.sqrt(gx * gx + gy * gy + gz * gz)[..., None]
    elev = jnp.arccos(gz[..., None] / (npd + EPS))
    azim = jnp.arccos(gx[..., None] / (npd * jnp.sin(elev) + EPS))
    azim = jnp.where(gy[..., None] < 0.0, 2.0 * math.pi - azim, azim)
    hit_sky = dist >= (1e8 - 1)
    return dist, idx, hit_sky, npd, elev, azim


# speedup vs baseline: 1.9794x; 1.9794x over previous
import functools
import math

import jax
import jax.numpy as jnp
from jax import lax
from jax.experimental import pallas as pl
from jax.experimental.pallas import tpu as pltpu
from jax.experimental.pallas import tpu_sc as plsc

K = 8
NPTS = 10000
NRAY = 8
NCAND = 16
L = 16
NWORK = 32
BIG = 3.0e38
EPS = 1e-05
CONE2_LOOSE = 0.7498
ENC_BASE = 1.0e16
ENC_STEP = 1.1e9


def _sc_search_body(pxh, pyh, pzh, oxh, oyh, ozh, rdxh, rdyh, rdzh, c0h,
                    oidx_h, ogx_h, ogy_h, ogz_h,
                    px_v, py_v, pz_v, ox_v, oy_v, oz_v,
                    rdx_v, rdy_v, rdz_v, c0_v,
                    oi_v, ogx_v, ogy_v, ogz_v):
    nc = plsc.get_sparse_core_info().num_cores
    wid = lax.axis_index("s") * nc + lax.axis_index("c")
    f0 = wid * 16
    r0 = wid * 128

    pltpu.sync_copy(pxh, px_v)
    pltpu.sync_copy(pyh, py_v)
    pltpu.sync_copy(pzh, pz_v)
    pltpu.sync_copy(oxh.at[pl.ds(f0, 16)], ox_v)
    pltpu.sync_copy(oyh.at[pl.ds(f0, 16)], oy_v)
    pltpu.sync_copy(ozh.at[pl.ds(f0, 16)], oz_v)
    pltpu.sync_copy(rdxh.at[pl.ds(r0, 128)], rdx_v)
    pltpu.sync_copy(rdyh.at[pl.ds(r0, 128)], rdy_v)
    pltpu.sync_copy(rdzh.at[pl.ds(r0, 128)], rdz_v)
    pltpu.sync_copy(c0h.at[pl.ds(r0, 128)], c0_v)

    ila = lax.iota(jnp.int32, L)

    def origin_body(fl, carry):
        ox = ox_v[fl]
        oy = oy_v[fl]
        oz = oz_v[fl]
        rdxs = [rdx_v[fl * NRAY + r] for r in range(NRAY)]
        rdys = [rdy_v[fl * NRAY + r] for r in range(NRAY)]
        rdzs = [rdz_v[fl * NRAY + r] for r in range(NRAY)]
        c0s = [c0_v[fl * NRAY + r] for r in range(NRAY)]

        t0 = tuple(jnp.full((L,), BIG, jnp.float32) for _ in range(NRAY))
        ti0 = tuple(jnp.zeros((L,), jnp.int32) for _ in range(NRAY))

        def chunk_body(j, tcarry):
            ts, tis = tcarry
            base = pl.multiple_of(j * L, L)
            px = px_v[pl.ds(base, L)]
            py = py_v[pl.ds(base, L)]
            pz = pz_v[pl.ds(base, L)]
            ddx = px - ox
            ddy = py - oy
            ddz = pz - oz
            d2 = ddx * ddx + ddy * ddy + ddz * ddz
            thr2 = CONE2_LOOSE * d2
            idxi = ila + base
            enc = ENC_BASE + idxi.astype(jnp.float32) * ENC_STEP
            nts, ntis = [], []
            for r in range(NRAY):
                dot = rdxs[r] * px + rdys[r] * py + rdzs[r] * pz - c0s[r]
                t2 = dot * dot
                s = d2 - t2
                keep = (dot >= 0.0) & (t2 >= thr2)
                val = jnp.where(keep, s, enc)
                vk, vi = plsc.sort_key_val(val, idxi, descending=True)
                cmp = ts[r] <= vk
                tn = jnp.where(cmp, ts[r], vk)
                tin = jnp.where(cmp, tis[r], vi)
                sk, si = plsc.sort_key_val(tn, tin, descending=False)
                nts.append(sk)
                ntis.append(si)
            return (tuple(nts), tuple(ntis))

        ts, tis = lax.fori_loop(0, NPTS // L, chunk_body, (t0, ti0))

        for r in range(NRAY):
            row = fl * NRAY + r
            oi_v[row] = tis[r]
            ogx_v[row] = plsc.load_gather(px_v, [tis[r]])
            ogy_v[row] = plsc.load_gather(py_v, [tis[r]])
            ogz_v[row] = plsc.load_gather(pz_v, [tis[r]])
        return carry

    lax.fori_loop(0, 16, origin_body, 0)

    pltpu.sync_copy(oi_v, oidx_h.at[pl.ds(r0, 128)])
    pltpu.sync_copy(ogx_v, ogx_h.at[pl.ds(r0, 128)])
    pltpu.sync_copy(ogy_v, ogy_h.at[pl.ds(r0, 128)])
    pltpu.sync_copy(ogz_v, ogz_h.at[pl.ds(r0, 128)])


def _rescore_body(ro_ref, rd_ref, ci_ref, cx_ref, cy_ref, cz_ref,
                  dist_ref, idx_ref, npd_ref, elev_ref, azim_ref):
    ox = ro_ref[:, 0:1]
    oy = ro_ref[:, 1:2]
    oz = ro_ref[:, 2:3]
    dx = cx_ref[...] - ox
    dy = cy_ref[...] - oy
    dz = cz_ref[...] - oz
    dn = jnp.sqrt(dx * dx + dy * dy + dz * dz)
    m = jnp.maximum(dn, 1e-12)
    ux = dx / m
    uy = dy / m
    uz = dz / m
    cos = rd_ref[:, 0:1] * ux + rd_ref[:, 1:2] * uy + rd_ref[:, 2:3] * uz
    om = 1.0 - cos * cos
    sin = jnp.sqrt(jnp.maximum(om, 1e-12))
    proj = sin * dn
    val = jnp.where(cos < 0.866, 1e8, proj)

    ci = ci_ref[...]
    cx = cx_ref[...]
    cy = cy_ref[...]
    cz = cz_ref[...]
    mvs, ams, gxs, gys, gzs = [], [], [], [], []
    for _ in range(K):
        mv = jnp.min(val, axis=1, keepdims=True)
        eqm = val == mv
        am = jnp.min(jnp.where(eqm, ci, NPTS), axis=1, keepdims=True)
        sel = eqm & (ci == am)
        gxs.append(jnp.min(jnp.where(sel, cx, BIG), axis=1, keepdims=True))
        gys.append(jnp.min(jnp.where(sel, cy, BIG), axis=1, keepdims=True))
        gzs.append(jnp.min(jnp.where(sel, cz, BIG), axis=1, keepdims=True))
        mvs.append(mv)
        ams.append(am)
        val = jnp.where(sel, BIG, val)

    dist_ref[...] = jnp.concatenate(mvs, axis=1)
    idx_ref[...] = jnp.concatenate(ams, axis=1)
    gx = jnp.concatenate(gxs, axis=1) - ox
    gy = jnp.concatenate(gys, axis=1) - oy
    gz = jnp.concatenate(gzs, axis=1) - oz
    def _acos(x):
        return jnp.arctan2(jnp.sqrt(jnp.maximum(1.0 - x * x, 0.0)), x)

    npd = jnp.sqrt(gx * gx + gy * gy + gz * gz)
    elev = _acos(gz / (npd + EPS))
    azim = _acos(gx / (npd * jnp.sin(elev) + EPS))
    azim = jnp.where(gy < 0.0, 2.0 * math.pi - azim, azim)
    npd_ref[...] = npd
    elev_ref[...] = elev
    azim_ref[...] = azim


def kernel(ray_o, ray_d, pts):
    FTR = ray_o.shape[0]
    nrays = FTR * NRAY

    rdn = ray_d / jnp.maximum(jnp.linalg.norm(ray_d, axis=-1, keepdims=True), 1e-12)
    rdf = rdn.reshape(nrays, 3)
    ro_ray = jnp.repeat(ray_o, NRAY, axis=0)
    c0 = jnp.sum(rdf * ro_ray, axis=-1)

    px = pts[:, 0]
    py = pts[:, 1]
    pz = pts[:, 2]
    oxs = jnp.broadcast_to(ray_o[:, 0:1], (FTR, L))
    oys = jnp.broadcast_to(ray_o[:, 1:2], (FTR, L))
    ozs = jnp.broadcast_to(ray_o[:, 2:3], (FTR, L))
    rdxs = jnp.broadcast_to(rdf[:, 0:1], (nrays, L))
    rdys = jnp.broadcast_to(rdf[:, 1:2], (nrays, L))
    rdzs = jnp.broadcast_to(rdf[:, 2:3], (nrays, L))
    c0s = jnp.broadcast_to(c0[:, None], (nrays, L))

    mesh = plsc.VectorSubcoreMesh(core_axis_name="c", subcore_axis_name="s")
    sc_search = pl.kernel(
        _sc_search_body,
        out_type=[
            jax.ShapeDtypeStruct((nrays, NCAND), jnp.int32),
            jax.ShapeDtypeStruct((nrays, NCAND), jnp.float32),
            jax.ShapeDtypeStruct((nrays, NCAND), jnp.float32),
            jax.ShapeDtypeStruct((nrays, NCAND), jnp.float32),
        ],
        mesh=mesh,
        compiler_params=pltpu.CompilerParams(
            needs_layout_passes=False, use_tc_tiling_on_sc=False),
        scratch_types=[
            pltpu.VMEM((NPTS,), jnp.float32),
            pltpu.VMEM((NPTS,), jnp.float32),
            pltpu.VMEM((NPTS,), jnp.float32),
            pltpu.VMEM((16, L), jnp.float32),
            pltpu.VMEM((16, L), jnp.float32),
            pltpu.VMEM((16, L), jnp.float32),
            pltpu.VMEM((128, L), jnp.float32),
            pltpu.VMEM((128, L), jnp.float32),
            pltpu.VMEM((128, L), jnp.float32),
            pltpu.VMEM((128, L), jnp.float32),
            pltpu.VMEM((128, NCAND), jnp.int32),
            pltpu.VMEM((128, NCAND), jnp.float32),
            pltpu.VMEM((128, NCAND), jnp.float32),
            pltpu.VMEM((128, NCAND), jnp.float32),
        ],
    )
    cidx, cgx, cgy, cgz = sc_search(px, py, pz, oxs, oys, ozs,
                                    rdxs, rdys, rdzs, c0s)

    ro4 = jnp.pad(ro_ray, ((0, 0), (0, 1)))
    rd4 = jnp.pad(rdf, ((0, 0), (0, 1)))
    oshape = jax.ShapeDtypeStruct((nrays, K), jnp.float32)
    dist, idx, npd, elev, azim = pl.pallas_call(
        _rescore_body,
        out_shape=(oshape, jax.ShapeDtypeStruct((nrays, K), jnp.int32),
                   oshape, oshape, oshape),
    )(ro4, rd4, cidx, cgx, cgy, cgz)

    dist = dist.reshape(FTR, NRAY, K)
    idx = idx.reshape(FTR, NRAY, K)
    npd = npd.reshape(FTR, NRAY, K)[..., None]
    elev = elev.reshape(FTR, NRAY, K)[..., None]
    azim = azim.reshape(FTR, NRAY, K)[..., None]
    hit_sky = dist >= (1e8 - 1)
    return dist, idx, hit_sky, npd, elev, azim


# speedup vs baseline: 2.0039x; 1.0123x over previous
import functools
import math

import jax
import jax.numpy as jnp
from jax import lax
from jax.experimental import pallas as pl
from jax.experimental.pallas import tpu as pltpu
from jax.experimental.pallas import tpu_sc as plsc

K = 8
NPTS = 10000
NRAY = 8
NCAND = 16
L = 16
NWORK = 32
BIG = 3.0e38
EPS = 1e-05
CONE2_LOOSE = 0.7498
ENC_BASE = 1.0e16
ENC_STEP = 1.1e9


def _sc_search_body(pxh, pyh, pzh, oxh, oyh, ozh, rdxh, rdyh, rdzh, c0h,
                    oidx_h, ogx_h, ogy_h, ogz_h,
                    px_v, py_v, pz_v, ox_v, oy_v, oz_v,
                    rdx_v, rdy_v, rdz_v, c0_v,
                    oi_v, ogx_v, ogy_v, ogz_v):
    nc = plsc.get_sparse_core_info().num_cores
    wid = lax.axis_index("s") * nc + lax.axis_index("c")
    f0 = wid * 16
    r0 = wid * 128

    pltpu.sync_copy(pxh, px_v)
    pltpu.sync_copy(pyh, py_v)
    pltpu.sync_copy(pzh, pz_v)
    pltpu.sync_copy(oxh.at[pl.ds(f0, 16)], ox_v)
    pltpu.sync_copy(oyh.at[pl.ds(f0, 16)], oy_v)
    pltpu.sync_copy(ozh.at[pl.ds(f0, 16)], oz_v)
    pltpu.sync_copy(rdxh.at[pl.ds(r0, 128)], rdx_v)
    pltpu.sync_copy(rdyh.at[pl.ds(r0, 128)], rdy_v)
    pltpu.sync_copy(rdzh.at[pl.ds(r0, 128)], rdz_v)
    pltpu.sync_copy(c0h.at[pl.ds(r0, 128)], c0_v)

    ila = lax.iota(jnp.int32, L)

    def origin_body(fl, carry):
        ox = ox_v[fl]
        oy = oy_v[fl]
        oz = oz_v[fl]
        rdxs = [rdx_v[fl * NRAY + r] for r in range(NRAY)]
        rdys = [rdy_v[fl * NRAY + r] for r in range(NRAY)]
        rdzs = [rdz_v[fl * NRAY + r] for r in range(NRAY)]
        c0s = [c0_v[fl * NRAY + r] for r in range(NRAY)]

        t0 = tuple(jnp.full((L,), BIG, jnp.float32) for _ in range(NRAY))
        ti0 = tuple(jnp.zeros((L,), jnp.int32) for _ in range(NRAY))

        def chunk_body(j, tcarry):
            ts, tis = tcarry
            base = pl.multiple_of(j * L, L)
            px = px_v[pl.ds(base, L)]
            py = py_v[pl.ds(base, L)]
            pz = pz_v[pl.ds(base, L)]
            ddx = px - ox
            ddy = py - oy
            ddz = pz - oz
            d2 = ddx * ddx + ddy * ddy + ddz * ddz
            thr2 = CONE2_LOOSE * d2
            idxi = ila + base
            enc = ENC_BASE + idxi.astype(jnp.float32) * ENC_STEP
            nts, ntis = [], []
            for r in range(NRAY):
                dot = rdxs[r] * px + rdys[r] * py + rdzs[r] * pz - c0s[r]
                t2 = dot * dot
                s = d2 - t2
                keep = (dot >= 0.0) & (t2 >= thr2)
                val = jnp.where(keep, s, enc)
                vk, vi = plsc.sort_key_val(val, idxi, descending=True)
                cmp = ts[r] <= vk
                tn = jnp.where(cmp, ts[r], vk)
                tin = jnp.where(cmp, tis[r], vi)
                sk, si = plsc.sort_key_val(tn, tin, descending=False)
                nts.append(sk)
                ntis.append(si)
            return (tuple(nts), tuple(ntis))

        ts, tis = lax.fori_loop(0, NPTS // L, chunk_body, (t0, ti0))

        for r in range(NRAY):
            row = fl * NRAY + r
            oi_v[row] = tis[r]
            ogx_v[row] = plsc.load_gather(px_v, [tis[r]])
            ogy_v[row] = plsc.load_gather(py_v, [tis[r]])
            ogz_v[row] = plsc.load_gather(pz_v, [tis[r]])
        return carry

    lax.fori_loop(0, 16, origin_body, 0)

    pltpu.sync_copy(oi_v, oidx_h.at[pl.ds(r0, 128)])
    pltpu.sync_copy(ogx_v, ogx_h.at[pl.ds(r0, 128)])
    pltpu.sync_copy(ogy_v, ogy_h.at[pl.ds(r0, 128)])
    pltpu.sync_copy(ogz_v, ogz_h.at[pl.ds(r0, 128)])


def _rescore_body(ro_ref, rd_ref, ci_ref, cx_ref, cy_ref, cz_ref,
                  dist_ref, idx_ref, gx_ref, gy_ref, gz_ref):
    ox = ro_ref[:, 0:1]
    oy = ro_ref[:, 1:2]
    oz = ro_ref[:, 2:3]
    dx = cx_ref[...] - ox
    dy = cy_ref[...] - oy
    dz = cz_ref[...] - oz
    dn = jnp.sqrt(dx * dx + dy * dy + dz * dz)
    m = jnp.maximum(dn, 1e-12)
    ux = dx / m
    uy = dy / m
    uz = dz / m
    cos = rd_ref[:, 0:1] * ux + rd_ref[:, 1:2] * uy + rd_ref[:, 2:3] * uz
    om = 1.0 - cos * cos
    sin = jnp.sqrt(jnp.maximum(om, 1e-12))
    proj = sin * dn
    val = jnp.where(cos < 0.866, 1e8, proj)

    ci = ci_ref[...]
    cx = cx_ref[...]
    cy = cy_ref[...]
    cz = cz_ref[...]
    mvs, ams, gxs, gys, gzs = [], [], [], [], []
    for _ in range(K):
        mv = jnp.min(val, axis=1, keepdims=True)
        eqm = val == mv
        am = jnp.min(jnp.where(eqm, ci, NPTS), axis=1, keepdims=True)
        sel = eqm & (ci == am)
        gxs.append(jnp.min(jnp.where(sel, cx, BIG), axis=1, keepdims=True))
        gys.append(jnp.min(jnp.where(sel, cy, BIG), axis=1, keepdims=True))
        gzs.append(jnp.min(jnp.where(sel, cz, BIG), axis=1, keepdims=True))
        mvs.append(mv)
        ams.append(am)
        val = jnp.where(sel, BIG, val)

    dist_ref[...] = jnp.concatenate(mvs, axis=1)
    idx_ref[...] = jnp.concatenate(ams, axis=1)
    gx_ref[...] = jnp.concatenate(gxs, axis=1) - ox
    gy_ref[...] = jnp.concatenate(gys, axis=1) - oy
    gz_ref[...] = jnp.concatenate(gzs, axis=1) - oz


def kernel(ray_o, ray_d, pts):
    FTR = ray_o.shape[0]
    nrays = FTR * NRAY

    rdn = ray_d / jnp.maximum(jnp.linalg.norm(ray_d, axis=-1, keepdims=True), 1e-12)
    rdf = rdn.reshape(nrays, 3)
    ro_ray = jnp.repeat(ray_o, NRAY, axis=0)
    c0 = jnp.sum(rdf * ro_ray, axis=-1)

    px = pts[:, 0]
    py = pts[:, 1]
    pz = pts[:, 2]
    oxs = jnp.broadcast_to(ray_o[:, 0:1], (FTR, L))
    oys = jnp.broadcast_to(ray_o[:, 1:2], (FTR, L))
    ozs = jnp.broadcast_to(ray_o[:, 2:3], (FTR, L))
    rdxs = jnp.broadcast_to(rdf[:, 0:1], (nrays, L))
    rdys = jnp.broadcast_to(rdf[:, 1:2], (nrays, L))
    rdzs = jnp.broadcast_to(rdf[:, 2:3], (nrays, L))
    c0s = jnp.broadcast_to(c0[:, None], (nrays, L))

    mesh = plsc.VectorSubcoreMesh(core_axis_name="c", subcore_axis_name="s")
    sc_search = pl.kernel(
        _sc_search_body,
        out_type=[
            jax.ShapeDtypeStruct((nrays, NCAND), jnp.int32),
            jax.ShapeDtypeStruct((nrays, NCAND), jnp.float32),
            jax.ShapeDtypeStruct((nrays, NCAND), jnp.float32),
            jax.ShapeDtypeStruct((nrays, NCAND), jnp.float32),
        ],
        mesh=mesh,
        compiler_params=pltpu.CompilerParams(
            needs_layout_passes=False, use_tc_tiling_on_sc=False),
        scratch_types=[
            pltpu.VMEM((NPTS,), jnp.float32),
            pltpu.VMEM((NPTS,), jnp.float32),
            pltpu.VMEM((NPTS,), jnp.float32),
            pltpu.VMEM((16, L), jnp.float32),
            pltpu.VMEM((16, L), jnp.float32),
            pltpu.VMEM((16, L), jnp.float32),
            pltpu.VMEM((128, L), jnp.float32),
            pltpu.VMEM((128, L), jnp.float32),
            pltpu.VMEM((128, L), jnp.float32),
            pltpu.VMEM((128, L), jnp.float32),
            pltpu.VMEM((128, NCAND), jnp.int32),
            pltpu.VMEM((128, NCAND), jnp.float32),
            pltpu.VMEM((128, NCAND), jnp.float32),
            pltpu.VMEM((128, NCAND), jnp.float32),
        ],
    )
    cidx, cgx, cgy, cgz = sc_search(px, py, pz, oxs, oys, ozs,
                                    rdxs, rdys, rdzs, c0s)

    ro4 = jnp.pad(ro_ray, ((0, 0), (0, 1)))
    rd4 = jnp.pad(rdf, ((0, 0), (0, 1)))
    oshape = jax.ShapeDtypeStruct((nrays, K), jnp.float32)
    dist, idx, gx, gy, gz = pl.pallas_call(
        _rescore_body,
        out_shape=(oshape, jax.ShapeDtypeStruct((nrays, K), jnp.int32),
                   oshape, oshape, oshape),
    )(ro4, rd4, cidx, cgx, cgy, cgz)

    dist = dist.reshape(FTR, NRAY, K)
    idx = idx.reshape(FTR, NRAY, K)
    gx = gx.reshape(FTR, NRAY, K)
    gy = gy.reshape(FTR, NRAY, K)
    gz = gz.reshape(FTR, NRAY, K)

    npd = jnp.sqrt(gx * gx + gy * gy + gz * gz)[..., None]
    elev = jnp.arccos(gz[..., None] / (npd + EPS))
    azim = jnp.arccos(gx[..., None] / (npd * jnp.sin(elev) + EPS))
    azim = jnp.where(gy[..., None] < 0.0, 2.0 * math.pi - azim, azim)
    hit_sky = dist >= (1e8 - 1)
    return dist, idx, hit_sky, npd, elev, azim
